# Initial kernel scaffold; baseline (speedup 1.0000x reference)
import jax
import jax.numpy as jnp
from jax.experimental import pallas as pl


def kernel(x_dense, x_sparse, W_sparse, b_sparse, W_dense, b_dense):
    raise NotImplementedError("write your pallas kernel here")


# speedup vs baseline: 1.0633x; 1.0633x over previous
import functools

import jax
import jax.numpy as jnp
from jax import lax
from jax.experimental import pallas as pl
from jax.experimental.pallas import tpu as pltpu
from jax.experimental.pallas import tpu_sc as plsc

NUM_CORES = 2
NUM_SUBCORES = 16
NW = NUM_CORES * NUM_SUBCORES
LANES = 16

BATCH = 16384
NFIELD = 26
FIELD_SIZE = 100000
DDIM = 13
BPW = BATCH // NW
CHUNKS = BPW // LANES


def _sc_body(xs_hbm, xd_hbm, table_hbm, wd_hbm, bs_hbm, bd_hbm, out_hbm,
             xs_v, idx_v, vals_v, xd_v, wd_v, bs_v, bd_v, acc_v, sem):
  wid = lax.axis_index("s") * NUM_CORES + lax.axis_index("c")
  base = wid * BPW

  pltpu.sync_copy(xs_hbm.at[pl.ds(base * NFIELD, BPW * NFIELD)], xs_v)
  pltpu.sync_copy(xd_hbm.at[pl.ds(base * DDIM, BPW * DDIM)], xd_v)
  pltpu.sync_copy(wd_hbm, wd_v.at[pl.ds(0, DDIM)])
  pltpu.sync_copy(bs_hbm, bs_v.at[pl.ds(0, 1)])
  pltpu.sync_copy(bd_hbm, bd_v.at[pl.ds(0, 1)])

  iota = lax.iota(jnp.int32, LANES)
  iota_f = iota * NFIELD
  iota_d = iota * DDIM

  def idx_body(c, carry):
    pos0 = c * (LANES * NFIELD)
    for f in range(NFIELD):
      g = iota_f + (pos0 + f)
      v = plsc.load_gather(xs_v, [g])
      idx_v[pl.ds(f * BPW + c * LANES, LANES)] = v + (f * FIELD_SIZE)
    return carry

  lax.fori_loop(0, CHUNKS, idx_body, 0)

  pltpu.async_copy(table_hbm.at[idx_v], vals_v, sem).wait()

  bs_vec = bs_v[pl.ds(0, LANES)]
  bd_vec = bd_v[pl.ds(0, LANES)]
  wd_vec = wd_v[pl.ds(0, LANES)]
  bias = bs_vec[0] + bd_vec[0]
  wds = [wd_vec[d] for d in range(DDIM)]

  def red_body(c, carry):
    off = c * LANES
    acc = jnp.zeros((LANES,), jnp.float32) + bias
    for f in range(NFIELD):
      acc = acc + vals_v[pl.ds(f * BPW + off, LANES)]
    dpos0 = c * (LANES * DDIM)
    for d in range(DDIM):
      xv = plsc.load_gather(xd_v, [iota_d + (dpos0 + d)])
      acc = acc + xv * wds[d]
    acc_v[pl.ds(off, LANES)] = acc
    return carry

  lax.fori_loop(0, CHUNKS, red_body, 0)

  pltpu.sync_copy(acc_v, out_hbm.at[pl.ds(base, BPW)])


@jax.jit
def _lrmodel_sc(xs, xd, table, wd, bs, bd):
  f = pl.kernel(
      _sc_body,
      out_type=jax.ShapeDtypeStruct((BATCH,), jnp.float32),
      mesh=plsc.VectorSubcoreMesh(
          core_axis_name="c", subcore_axis_name="s",
          num_cores=NUM_CORES, num_subcores=NUM_SUBCORES),
      scratch_types=[
          pltpu.VMEM((BPW * NFIELD,), jnp.int32),
          pltpu.VMEM((BPW * NFIELD,), jnp.int32),
          pltpu.VMEM((BPW * NFIELD,), jnp.float32),
          pltpu.VMEM((BPW * DDIM,), jnp.float32),
          pltpu.VMEM((LANES,), jnp.float32),
          pltpu.VMEM((LANES,), jnp.float32),
          pltpu.VMEM((LANES,), jnp.float32),
          pltpu.VMEM((BPW,), jnp.float32),
          pltpu.SemaphoreType.DMA,
      ],
      compiler_params=pltpu.CompilerParams(needs_layout_passes=False),
  )
  return f(xs, xd, table, wd, bs, bd)


def kernel(x_dense, x_sparse, W_sparse, b_sparse, W_dense, b_dense):
  xs = x_sparse.astype(jnp.int32).reshape(-1)
  xd = x_dense.reshape(-1)
  table = W_sparse.reshape(-1)
  wd = W_dense.reshape(-1)
  out = _lrmodel_sc(xs, xd, table, wd, b_sparse, b_dense)
  return out.reshape(BATCH, 1)


# speedup vs baseline: 1.2615x; 1.1864x over previous
import jax
import jax.numpy as jnp
from jax import lax
from jax.experimental import pallas as pl
from jax.experimental.pallas import tpu as pltpu
from jax.experimental.pallas import tpu_sc as plsc

NUM_CORES = 2
NUM_SUBCORES = 16
NW = NUM_CORES * NUM_SUBCORES
LANES = 16

BATCH = 16384
NFIELD = 26
FIELD_SIZE = 100000
DDIM = 13
BPW = BATCH // NW
CHUNKS = BPW // LANES


def _sc_body(xs_hbm, xd_hbm, table_hbm, wd_hbm, bs_hbm, bd_hbm, out_hbm,
             xs_v, idx_v, vals_v, xd_v, wd_v, bs_v, bd_v, acc_v, sem):
  wid = lax.axis_index("s") * NUM_CORES + lax.axis_index("c")
  base = wid * BPW

  pltpu.sync_copy(xs_hbm.at[:, pl.ds(base, BPW)], xs_v)
  pltpu.sync_copy(xd_hbm.at[:, pl.ds(base, BPW)], xd_v)
  pltpu.sync_copy(wd_hbm, wd_v.at[pl.ds(0, DDIM)])
  pltpu.sync_copy(bs_hbm, bs_v.at[pl.ds(0, 1)])
  pltpu.sync_copy(bd_hbm, bd_v.at[pl.ds(0, 1)])

  def idx_body(c, carry):
    off = c * LANES
    for f in range(NFIELD):
      idx_v[pl.ds(f * BPW + off, LANES)] = (
          xs_v[f, pl.ds(off, LANES)] + (f * FIELD_SIZE))
    return carry

  lax.fori_loop(0, CHUNKS, idx_body, 0)

  pltpu.async_copy(table_hbm.at[idx_v], vals_v, sem).wait()

  bs_vec = bs_v[pl.ds(0, LANES)]
  bd_vec = bd_v[pl.ds(0, LANES)]
  wd_vec = wd_v[pl.ds(0, LANES)]
  bias = bs_vec[0] + bd_vec[0]
  wds = [wd_vec[d] for d in range(DDIM)]

  def red_body(c, carry):
    off = c * LANES
    acc = jnp.zeros((LANES,), jnp.float32) + bias
    for f in range(NFIELD):
      acc = acc + vals_v[pl.ds(f * BPW + off, LANES)]
    for d in range(DDIM):
      acc = acc + xd_v[d, pl.ds(off, LANES)] * wds[d]
    acc_v[pl.ds(off, LANES)] = acc
    return carry

  lax.fori_loop(0, CHUNKS, red_body, 0)

  pltpu.sync_copy(acc_v, out_hbm.at[pl.ds(base, BPW)])


@jax.jit
def _lrmodel_sc(xs, xd, table, wd, bs, bd):
  f = pl.kernel(
      _sc_body,
      out_type=jax.ShapeDtypeStruct((BATCH,), jnp.float32),
      mesh=plsc.VectorSubcoreMesh(
          core_axis_name="c", subcore_axis_name="s",
          num_cores=NUM_CORES, num_subcores=NUM_SUBCORES),
      scratch_types=[
          pltpu.VMEM((NFIELD, BPW), jnp.int32),
          pltpu.VMEM((BPW * NFIELD,), jnp.int32),
          pltpu.VMEM((BPW * NFIELD,), jnp.float32),
          pltpu.VMEM((DDIM, BPW), jnp.float32),
          pltpu.VMEM((LANES,), jnp.float32),
          pltpu.VMEM((LANES,), jnp.float32),
          pltpu.VMEM((LANES,), jnp.float32),
          pltpu.VMEM((BPW,), jnp.float32),
          pltpu.SemaphoreType.DMA,
      ],
      compiler_params=pltpu.CompilerParams(needs_layout_passes=False),
  )
  return f(xs, xd, table, wd, bs, bd)


def kernel(x_dense, x_sparse, W_sparse, b_sparse, W_dense, b_dense):
  xs = x_sparse.astype(jnp.int32).T
  xd = x_dense.T
  table = W_sparse.reshape(-1)
  wd = W_dense.reshape(-1)
  out = _lrmodel_sc(xs, xd, table, wd, b_sparse, b_dense)
  return out.reshape(BATCH, 1)


# speedup vs baseline: 1.4871x; 1.1788x over previous
import jax
import jax.numpy as jnp
from jax import lax
from jax.experimental import pallas as pl
from jax.experimental.pallas import tpu as pltpu
from jax.experimental.pallas import tpu_sc as plsc

NUM_CORES = 2
NUM_SUBCORES = 16
NW = NUM_CORES * NUM_SUBCORES
LANES = 16

BATCH = 16384
NFIELD = 26
FIELD_SIZE = 100000
DDIM = 13
BPW = BATCH // NW
CHUNKS = BPW // LANES

SPLIT_F = 20
SPLIT_ROW = SPLIT_F * FIELD_SIZE
NF_A = SPLIT_F
NF_B = NFIELD - SPLIT_F

GROUPS_A = ((0, 7), (7, 7), (14, 6))
GROUPS_B = ((20, 6),)


def _stage_core(xs_v, idx_v, vals_v, acc_v, table_hbm, gsems, groups,
                row_base, init):
  gathers = []
  for g, (f0, nf) in enumerate(groups):
    lf0 = f0 - groups[0][0]

    def idx_body(c, carry, f0=f0, nf=nf, lf0=lf0):
      off = c * LANES
      for j in range(nf):
        f = f0 + j
        idx_v[pl.ds((lf0 + j) * BPW + off, LANES)] = (
            xs_v[f, pl.ds(off, LANES)] + (f * FIELD_SIZE - row_base))
      return carry

    lax.fori_loop(0, CHUNKS, idx_body, 0)
    sl = pl.ds(lf0 * BPW, nf * BPW)
    gathers.append(
        pltpu.async_copy(table_hbm.at[idx_v.at[sl]], vals_v.at[sl], gsems[g]))

  init()

  for g, (f0, nf) in enumerate(groups):
    lf0 = f0 - groups[0][0]
    gathers[g].wait()

    def red_body(c, carry, nf=nf, lf0=lf0):
      off = c * LANES
      acc = acc_v[pl.ds(off, LANES)]
      for j in range(nf):
        acc = acc + vals_v[pl.ds((lf0 + j) * BPW + off, LANES)]
      acc_v[pl.ds(off, LANES)] = acc
      return carry

    lax.fori_loop(0, CHUNKS, red_body, 0)


def _sc_a_body(xs_hbm, table_hbm, out_hbm,
               xs_v, idx_v, vals_v, acc_v, sem_xs, sem_g0, sem_g1, sem_g2):
  wid = lax.axis_index("s") * NUM_CORES + lax.axis_index("c")
  base = wid * BPW
  pltpu.async_copy(xs_hbm.at[:, pl.ds(base, BPW)], xs_v, sem_xs).wait()

  def init():
    def z_body(c, carry):
      acc_v[pl.ds(c * LANES, LANES)] = jnp.zeros((LANES,), jnp.float32)
      return carry
    lax.fori_loop(0, CHUNKS, z_body, 0)

  _stage_core(xs_v, idx_v, vals_v, acc_v, table_hbm,
              [sem_g0, sem_g1, sem_g2], GROUPS_A, 0, init)
  pltpu.sync_copy(acc_v, out_hbm.at[pl.ds(base, BPW)])


def _sc_b_body(xs_hbm, xd_hbm, table_hbm, wd_hbm, bs_hbm, bd_hbm, part_hbm,
               out_hbm, xs_v, idx_v, vals_v, xd_v, wd_v, bs_v, bd_v, part_v,
               acc_v, sem_xs, sem_xd, sem_p, sem_g0):
  wid = lax.axis_index("s") * NUM_CORES + lax.axis_index("c")
  base = wid * BPW
  xs_cp = pltpu.async_copy(xs_hbm.at[:, pl.ds(base, BPW)], xs_v, sem_xs)
  xd_cp = pltpu.async_copy(xd_hbm.at[:, pl.ds(base, BPW)], xd_v, sem_xd)
  pt_cp = pltpu.async_copy(part_hbm.at[pl.ds(base, BPW)], part_v, sem_p)
  pltpu.sync_copy(wd_hbm, wd_v.at[pl.ds(0, DDIM)])
  pltpu.sync_copy(bs_hbm, bs_v.at[pl.ds(0, 1)])
  pltpu.sync_copy(bd_hbm, bd_v.at[pl.ds(0, 1)])
  xs_cp.wait()

  bs_vec = bs_v[pl.ds(0, LANES)]
  bd_vec = bd_v[pl.ds(0, LANES)]
  wd_vec = wd_v[pl.ds(0, LANES)]
  bias = bs_vec[0] + bd_vec[0]
  wds = [wd_vec[d] for d in range(DDIM)]

  def init():
    xd_cp.wait()
    pt_cp.wait()

    def dense_body(c, carry):
      off = c * LANES
      acc = part_v[pl.ds(off, LANES)] + bias
      for d in range(DDIM):
        acc = acc + xd_v[d, pl.ds(off, LANES)] * wds[d]
      acc_v[pl.ds(off, LANES)] = acc
      return carry

    lax.fori_loop(0, CHUNKS, dense_body, 0)

  _stage_core(xs_v, idx_v, vals_v, acc_v, table_hbm,
              [sem_g0], GROUPS_B, SPLIT_ROW, init)
  pltpu.sync_copy(acc_v, out_hbm.at[pl.ds(base, BPW)])


_MESH = dict(core_axis_name="c", subcore_axis_name="s",
             num_cores=NUM_CORES, num_subcores=NUM_SUBCORES)


@jax.jit
def _lrmodel_sc(xs, xd, table_a, table_b, wd, bs, bd):
  fa = pl.kernel(
      _sc_a_body,
      out_type=jax.ShapeDtypeStruct((BATCH,), jnp.float32),
      mesh=plsc.VectorSubcoreMesh(**_MESH),
      scratch_types=[
          pltpu.VMEM((NFIELD, BPW), jnp.int32),
          pltpu.VMEM((BPW * NF_A,), jnp.int32),
          pltpu.VMEM((BPW * NF_A,), jnp.float32),
          pltpu.VMEM((BPW,), jnp.float32),
          pltpu.SemaphoreType.DMA,
          pltpu.SemaphoreType.DMA,
          pltpu.SemaphoreType.DMA,
          pltpu.SemaphoreType.DMA,
      ],
      compiler_params=pltpu.CompilerParams(needs_layout_passes=False),
  )
  partial = fa(xs, table_a)
  fb = pl.kernel(
      _sc_b_body,
      out_type=jax.ShapeDtypeStruct((BATCH,), jnp.float32),
      mesh=plsc.VectorSubcoreMesh(**_MESH),
      scratch_types=[
          pltpu.VMEM((NFIELD, BPW), jnp.int32),
          pltpu.VMEM((BPW * NF_B,), jnp.int32),
          pltpu.VMEM((BPW * NF_B,), jnp.float32),
          pltpu.VMEM((DDIM, BPW), jnp.float32),
          pltpu.VMEM((LANES,), jnp.float32),
          pltpu.VMEM((LANES,), jnp.float32),
          pltpu.VMEM((LANES,), jnp.float32),
          pltpu.VMEM((BPW,), jnp.float32),
          pltpu.VMEM((BPW,), jnp.float32),
          pltpu.SemaphoreType.DMA,
          pltpu.SemaphoreType.DMA,
          pltpu.SemaphoreType.DMA,
          pltpu.SemaphoreType.DMA,
      ],
      compiler_params=pltpu.CompilerParams(needs_layout_passes=False),
  )
  return fb(xs, xd, table_b, wd, bs, bd, partial)


def kernel(x_dense, x_sparse, W_sparse, b_sparse, W_dense, b_dense):
  xs = x_sparse.astype(jnp.int32).T
  xd = x_dense.T
  table_a = W_sparse[:SPLIT_ROW].reshape(-1)
  table_b = W_sparse[SPLIT_ROW:].reshape(-1)
  wd = W_dense.reshape(-1)
  out = _lrmodel_sc(xs, xd, table_a, table_b, wd, b_sparse, b_dense)
  return out.reshape(BATCH, 1)
